# Initial kernel scaffold; baseline (speedup 1.0000x reference)
import jax
import jax.numpy as jnp
from jax.experimental import pallas as pl


def kernel(x, edge_index, W_l, W_r, att, bias, W1, b1, W2, b2, gamma, beta):
    raise NotImplementedError("write your pallas kernel here")


# speedup vs baseline: 8.5121x; 8.5121x over previous
import functools

import jax
import jax.numpy as jnp
from jax.experimental import pallas as pl
from jax.experimental.pallas import tpu as pltpu

N = 10000
E = 320000
D = 128
H = 4
DH = D // H

ROW_BLK = 1000


def _proj_body(x_ref, wl_ref, wr_ref, xl_ref, xr_ref):
    x = x_ref[...]
    xl_ref[...] = jnp.dot(x, wl_ref[...], preferred_element_type=jnp.float32)
    xr_ref[...] = jnp.dot(x, wr_ref[...], preferred_element_type=jnp.float32)


@jax.jit
def _proj(x, W_l, W_r):
    grid = (N // ROW_BLK,)
    return pl.pallas_call(
        _proj_body,
        grid=grid,
        in_specs=[
            pl.BlockSpec((ROW_BLK, D), lambda i: (i, 0)),
            pl.BlockSpec((D, D), lambda i: (0, 0)),
            pl.BlockSpec((D, D), lambda i: (0, 0)),
        ],
        out_specs=[
            pl.BlockSpec((ROW_BLK, D), lambda i: (i, 0)),
            pl.BlockSpec((ROW_BLK, D), lambda i: (i, 0)),
        ],
        out_shape=[
            jax.ShapeDtypeStruct((N, D), jnp.float32),
            jax.ShapeDtypeStruct((N, D), jnp.float32),
        ],
    )(x, W_l, W_r)


def _post_body(num_ref, den_ref, bias_ref, w1_ref, b1_ref, w2_ref, b2_ref,
               g_ref, bt_ref, y_ref):
    num = num_ref[...]
    den = den_ref[...]
    den_full = jnp.repeat(den, DH, axis=1)
    h = num / (den_full + 1e-16) + bias_ref[...]
    t = jnp.maximum(jnp.dot(h, w1_ref[...], preferred_element_type=jnp.float32)
                    + b1_ref[...], 0.0)
    y = jnp.dot(t, w2_ref[...], preferred_element_type=jnp.float32) + b2_ref[...] + h
    mean = jnp.mean(y, axis=-1, keepdims=True)
    yc = y - mean
    var = jnp.mean(yc * yc, axis=-1, keepdims=True)
    y_ref[...] = yc * jax.lax.rsqrt(var + 1e-6) * g_ref[...] + bt_ref[...]


@jax.jit
def _post(num, den, bias, W1, b1, W2, b2, gamma, beta):
    grid = (N // ROW_BLK,)
    row = lambda i: (i, 0)
    fixed = lambda i: (0, 0)
    y = pl.pallas_call(
        _post_body,
        grid=grid,
        in_specs=[
            pl.BlockSpec((ROW_BLK, D), row),
            pl.BlockSpec((ROW_BLK, H), row),
            pl.BlockSpec((1, D), fixed),
            pl.BlockSpec((D, D), fixed),
            pl.BlockSpec((1, D), fixed),
            pl.BlockSpec((D, D), fixed),
            pl.BlockSpec((1, D), fixed),
            pl.BlockSpec((1, D), fixed),
            pl.BlockSpec((1, D), fixed),
        ],
        out_specs=pl.BlockSpec((ROW_BLK, D), row),
        out_shape=jax.ShapeDtypeStruct((N, D), jnp.float32),
    )(num, den, bias.reshape(1, D), W1, b1.reshape(1, D), W2, b2.reshape(1, D),
      gamma.reshape(1, D), beta.reshape(1, D))
    return y[None, :, :]


def _edge_phase(xl, xr, src, dst, att):
    xj = xl[src].reshape(-1, H, DH)
    xi = xr[dst].reshape(-1, H, DH)
    e = xi + xj
    e = jnp.where(e > 0, e, 0.2 * e)
    logit = jnp.einsum('ehd,hd->eh', e, att)
    s = jnp.exp(logit)
    den = jax.ops.segment_sum(s, dst, num_segments=N)
    num = jax.ops.segment_sum((xj * s[..., None]).reshape(-1, D), dst,
                              num_segments=N)
    return num, den


def kernel(x, edge_index, W_l, W_r, att, bias, W1, b1, W2, b2, gamma, beta):
    xl, xr = _proj(x, W_l, W_r)
    loop = jnp.arange(N, dtype=jnp.int32)
    src = jnp.concatenate([edge_index[0].astype(jnp.int32), loop])
    dst = jnp.concatenate([edge_index[1].astype(jnp.int32), loop])
    num, den = _edge_phase(xl, xr, src, dst, att)
    return _post(num, den, bias, W1, b1, W2, b2, gamma, beta)


# speedup vs baseline: 12.0543x; 1.4161x over previous
import functools

import jax
import jax.numpy as jnp
from jax import lax
from jax.experimental import pallas as pl
from jax.experimental.pallas import tpu as pltpu
from jax.experimental.pallas import tpu_sc as plsc

N = 10000
E = 320000
D = 128
H = 4
DH = D // H

ROW_BLK = 1000

NC = 2
NS = 16
NW = NC * NS
NP = 10112
RPT = NP // NS
EN = E + N
C = 128
K = 81
EN_PAD = NW * K * C
IDX_ROWS = EN_PAD // 128


def _proj_body(x_ref, wl_ref, wr_ref, xl_ref, xr_ref):
    x = x_ref[...]
    xl_ref[...] = jnp.dot(x, wl_ref[...], preferred_element_type=jnp.float32)
    xr_ref[...] = jnp.dot(x, wr_ref[...], preferred_element_type=jnp.float32)


@jax.jit
def _proj(x, W_l, W_r):
    grid = (N // ROW_BLK,)
    return pl.pallas_call(
        _proj_body,
        grid=grid,
        in_specs=[
            pl.BlockSpec((ROW_BLK, D), lambda i: (i, 0)),
            pl.BlockSpec((D, D), lambda i: (0, 0)),
            pl.BlockSpec((D, D), lambda i: (0, 0)),
        ],
        out_specs=[
            pl.BlockSpec((ROW_BLK, D), lambda i: (i, 0)),
            pl.BlockSpec((ROW_BLK, D), lambda i: (i, 0)),
        ],
        out_shape=[
            jax.ShapeDtypeStruct((N, D), jnp.float32),
            jax.ShapeDtypeStruct((N, D), jnp.float32),
        ],
    )(x, W_l, W_r)


def _edge_body(xl_hbm, xr_hbm, srcm_hbm, dstm_hbm, att_hbm, znum_hbm, zden_hbm,
               onum_hbm, oden_hbm,
               acc_num, acc_den, xl_v, xr_v, den_v, src_v, dst_v,
               att_v, sem):
    c = lax.axis_index("c")
    s = lax.axis_index("s")
    wid = c * NS + s
    lo = pl.multiple_of(s * RPT, 8)

    pltpu.sync_copy(znum_hbm.at[pl.ds(lo, RPT)], acc_num.at[pl.ds(lo, RPT)])
    pltpu.sync_copy(zden_hbm.at[pl.ds(lo, RPT)], acc_den.at[pl.ds(lo, RPT)])
    pltpu.sync_copy(att_hbm, att_v)

    zero16 = jnp.zeros((16,), jnp.float32)

    def zden_body(i, carry):
        den_v[i, :] = zero16
        return carry

    lax.fori_loop(0, C, zden_body, 0)
    plsc.subcore_barrier()

    lane = lax.iota(jnp.int32, 16)

    def chunk_body(k, carry):
        g0 = wid * K + k
        pltpu.sync_copy(srcm_hbm.at[pl.ds(g0, 1)], src_v)
        pltpu.sync_copy(dstm_hbm.at[pl.ds(g0, 1)], dst_v)
        cp1 = pltpu.async_copy(xl_hbm.at[src_v.at[0]], xl_v, sem)
        cp2 = pltpu.async_copy(xr_hbm.at[dst_v.at[0]], xr_v, sem)
        cp1.wait()
        cp2.wait()

        def group_body(g, carry):
            eidx = g * 16 + lane
            for h in range(H):
                def logit_body(d, acc):
                    col = jnp.full((16,), h * DH, jnp.int32) + d
                    xlv = plsc.load_gather(xl_v, [eidx, col])
                    xrv = plsc.load_gather(xr_v, [eidx, col])
                    v = xlv + xrv
                    lr = jnp.maximum(v, 0.2 * v)
                    av = att_v[pl.ds(h * DH + d, 16)]
                    return acc + lr * av[0]

                logit = lax.fori_loop(0, DH, logit_body,
                                      jnp.zeros((16,), jnp.float32))
                sh = jnp.exp(logit)
                plsc.store_scatter(
                    den_v, [eidx, jnp.full((16,), h, jnp.int32)], sh)

                def num_body(d, carry2):
                    col = jnp.full((16,), h * DH, jnp.int32) + d
                    xlv = plsc.load_gather(xl_v, [eidx, col])
                    plsc.store_scatter(xl_v, [eidx, col], xlv * sh)
                    return carry2

                lax.fori_loop(0, DH, num_body, 0)
            return carry

        lax.fori_loop(0, C // 16, group_body, 0)

        pltpu.sync_copy(xl_v, acc_num.at[dst_v.at[0]], add=True)
        pltpu.sync_copy(den_v, acc_den.at[dst_v.at[0]], add=True)
        return carry

    lax.fori_loop(0, K, chunk_body, 0)
    plsc.subcore_barrier()

    pltpu.sync_copy(acc_num.at[pl.ds(lo, RPT)], onum_hbm.at[c, pl.ds(lo, RPT)])
    pltpu.sync_copy(acc_den.at[pl.ds(lo, RPT)], oden_hbm.at[c, pl.ds(lo, RPT)])


@jax.jit
def _edge_sc(xl_pad, xr_pad, srcm, dstm, att):
    znum = jnp.zeros((NP, D), jnp.float32)
    zden = jnp.zeros((NP, 16), jnp.float32)
    mesh = plsc.VectorSubcoreMesh(core_axis_name="c", subcore_axis_name="s")
    f = pl.kernel(
        _edge_body,
        out_type=[
            jax.ShapeDtypeStruct((NC, NP, D), jnp.float32),
            jax.ShapeDtypeStruct((NC, NP, 16), jnp.float32),
        ],
        mesh=mesh,
        scratch_types=[
            pltpu.VMEM_SHARED((NP, D), jnp.float32),
            pltpu.VMEM_SHARED((NP, 16), jnp.float32),
            pltpu.VMEM((C, D), jnp.float32),
            pltpu.VMEM((C, D), jnp.float32),
            pltpu.VMEM((C, 16), jnp.float32),
            pltpu.VMEM((1, C), jnp.int32),
            pltpu.VMEM((1, C), jnp.int32),
            pltpu.VMEM((D + 32,), jnp.float32),
            pltpu.SemaphoreType.DMA,
        ],
        compiler_params=pltpu.CompilerParams(needs_layout_passes=False,
                                             use_tc_tiling_on_sc=False),
    )
    return f(xl_pad, xr_pad, srcm, dstm, att, znum, zden)


def _post_body(num_ref, den_ref, bias_ref, w1_ref, b1_ref, w2_ref, b2_ref,
               g_ref, bt_ref, y_ref):
    num = num_ref[0] + num_ref[1]
    den = den_ref[0, :, :H] + den_ref[1, :, :H]
    den_full = jnp.repeat(den, DH, axis=1)
    h = num / (den_full + 1e-16) + bias_ref[...]
    t = jnp.maximum(jnp.dot(h, w1_ref[...], preferred_element_type=jnp.float32)
                    + b1_ref[...], 0.0)
    y = jnp.dot(t, w2_ref[...], preferred_element_type=jnp.float32) + b2_ref[...] + h
    mean = jnp.mean(y, axis=-1, keepdims=True)
    yc = y - mean
    var = jnp.mean(yc * yc, axis=-1, keepdims=True)
    y_ref[...] = yc * jax.lax.rsqrt(var + 1e-6) * g_ref[...] + bt_ref[...]


@jax.jit
def _post(onum, oden, bias, W1, b1, W2, b2, gamma, beta):
    grid = (N // ROW_BLK,)
    row3 = lambda i: (0, i, 0)
    fixed = lambda i: (0, 0)
    y = pl.pallas_call(
        _post_body,
        grid=grid,
        in_specs=[
            pl.BlockSpec((NC, ROW_BLK, D), row3),
            pl.BlockSpec((NC, ROW_BLK, 16), row3),
            pl.BlockSpec((1, D), fixed),
            pl.BlockSpec((D, D), fixed),
            pl.BlockSpec((1, D), fixed),
            pl.BlockSpec((D, D), fixed),
            pl.BlockSpec((1, D), fixed),
            pl.BlockSpec((1, D), fixed),
            pl.BlockSpec((1, D), fixed),
        ],
        out_specs=pl.BlockSpec((ROW_BLK, D), lambda i: (i, 0)),
        out_shape=jax.ShapeDtypeStruct((N, D), jnp.float32),
    )(onum, oden, bias.reshape(1, D), W1, b1.reshape(1, D), W2,
      b2.reshape(1, D), gamma.reshape(1, D), beta.reshape(1, D))
    return y[None, :, :]


def kernel(x, edge_index, W_l, W_r, att, bias, W1, b1, W2, b2, gamma, beta):
    xl, xr = _proj(x, W_l, W_r)
    pad_rows = jnp.zeros((NP - N, D), jnp.float32)
    xl_pad = jnp.concatenate([xl, pad_rows])
    xr_pad = jnp.concatenate([xr, pad_rows])
    loop = jnp.arange(N, dtype=jnp.int32)
    pad_idx = jnp.full((EN_PAD - EN,), N, jnp.int32)
    srcm = jnp.concatenate(
        [edge_index[0].astype(jnp.int32), loop, pad_idx]).reshape(IDX_ROWS, 128)
    dstm = jnp.concatenate(
        [edge_index[1].astype(jnp.int32), loop, pad_idx]).reshape(IDX_ROWS, 128)
    att_flat = jnp.concatenate([att.reshape(D), jnp.zeros((32,), jnp.float32)])
    onum, oden = _edge_sc(xl_pad, xr_pad, srcm, dstm, att_flat)
    return _post(onum, oden, bias, W1, b1, W2, b2, gamma, beta)
